# baseline (speedup 1.0000x reference)
import functools

import jax
import jax.numpy as jnp
from jax import lax
from jax.experimental import pallas as pl
from jax.experimental.pallas import tpu as pltpu
from jax.experimental.pallas import tpu_sc as plsc

_V = 1000000
_D = 32
_B = 16384
_NC = 2
_NS = 16
_L = 16
_NW = _NC * _NS
_BPW = _B // _NW
_CHUNK = 128
_NCHUNK = _BPW // _CHUNK
_NBLK = _BPW // _L
_BLK_PER_CHUNK = _CHUNK // _L


def _glove_body(center_hbm, outside_hbm, cooc_hbm, wt_hbm,
                wc_hbm, wo_hbm, bc_hbm, bo_hbm, out_hbm,
                cidx_v, oidx_v, ce_v, oe_v, bc_v, bo_v, cw_v, wt_v,
                part_v, sem):
    wid = lax.axis_index("s") * _NC + lax.axis_index("c")

    pltpu.sync_copy(center_hbm.at[wid], cidx_v)
    pltpu.sync_copy(outside_hbm.at[wid], oidx_v)
    pltpu.sync_copy(cooc_hbm.at[wid], cw_v)
    pltpu.sync_copy(wt_hbm.at[wid], wt_v)

    descs = []
    for j in range(_NCHUNK):
        descs.append(pltpu.async_copy(
            wc_hbm.at[cidx_v.at[j]], ce_v.at[pl.ds(j * _CHUNK, _CHUNK)], sem))
        descs.append(pltpu.async_copy(
            wo_hbm.at[oidx_v.at[j]], oe_v.at[pl.ds(j * _CHUNK, _CHUNK)], sem))
        descs.append(pltpu.async_copy(
            bc_hbm.at[cidx_v.at[j]], bc_v.at[pl.ds(j * _CHUNK, _CHUNK)], sem))
        descs.append(pltpu.async_copy(
            bo_hbm.at[oidx_v.at[j]], bo_v.at[pl.ds(j * _CHUNK, _CHUNK)], sem))
    for d in descs:
        d.wait()

    iota = lax.iota(jnp.int32, _L)

    def blk_step(blk, tot):
        rows = blk * _L + iota
        acc = jnp.zeros((_L,), jnp.float32)
        for d in range(_D):
            dvec = jnp.full((_L,), d, jnp.int32)
            a = plsc.load_gather(ce_v, [rows, dvec])
            b = plsc.load_gather(oe_v, [rows, dvec])
            acc = acc + a * b
        base = blk * _L
        bc16 = bc_v[pl.ds(base, _L)]
        bo16 = bo_v[pl.ds(base, _L)]
        cw16 = cw_v[pl.ds(base, _L)]
        wt16 = wt_v[pl.ds(base, _L)]
        err = acc + bc16 + bo16 - cw16
        return tot + wt16 * err * err

    tot = lax.fori_loop(0, _NBLK, blk_step, jnp.zeros((_L,), jnp.float32))
    part_v[...] = tot
    pltpu.sync_copy(part_v, out_hbm.at[wid])


@jax.jit
def _glove(center3, outside3, cooc2, wt2, wc, wo, bc1, bo1):
    mesh = plsc.VectorSubcoreMesh(core_axis_name="c", subcore_axis_name="s")
    run = functools.partial(
        pl.kernel,
        mesh=mesh,
        compiler_params=pltpu.CompilerParams(
            needs_layout_passes=False, use_tc_tiling_on_sc=False),
        out_type=jax.ShapeDtypeStruct((_NW, _L), jnp.float32),
        scratch_types=[
            pltpu.VMEM((_NCHUNK, _CHUNK), jnp.int32),
            pltpu.VMEM((_NCHUNK, _CHUNK), jnp.int32),
            pltpu.VMEM((_BPW, _D), jnp.float32),
            pltpu.VMEM((_BPW, _D), jnp.float32),
            pltpu.VMEM((_BPW,), jnp.float32),
            pltpu.VMEM((_BPW,), jnp.float32),
            pltpu.VMEM((_BPW,), jnp.float32),
            pltpu.VMEM((_BPW,), jnp.float32),
            pltpu.VMEM((_L,), jnp.float32),
            pltpu.SemaphoreType.DMA,
        ],
    )(_glove_body)
    return run(center3, outside3, cooc2, wt2, wc, wo, bc1, bo1)


def kernel(center, outside, coocs, weighting, W_center, W_outside,
           b_center, b_outside):
    center3 = center.reshape(_NW, _NCHUNK, _CHUNK).astype(jnp.int32)
    outside3 = outside.reshape(_NW, _NCHUNK, _CHUNK).astype(jnp.int32)
    cooc2 = coocs.reshape(_NW, _BPW)
    wt2 = weighting.reshape(_NW, _BPW)
    bc1 = b_center.reshape(_V)
    bo1 = b_outside.reshape(_V)
    partials = _glove(center3, outside3, cooc2, wt2,
                      W_center, W_outside, bc1, bo1)
    return jnp.sum(partials)
